# Initial kernel scaffold; baseline (speedup 1.0000x reference)
import jax
import jax.numpy as jnp
from jax.experimental import pallas as pl


def kernel(x, edge_index, W1_l, b1, W1_r, W2_l, b2, W2_r):
    raise NotImplementedError("write your pallas kernel here")


# speedup vs baseline: 9.7158x; 9.7158x over previous
import functools

import jax
import jax.numpy as jnp
from jax import lax
from jax.experimental import pallas as pl
from jax.experimental.pallas import tpu as pltpu
from jax.experimental.pallas import tpu_sc as plsc

N_NODES = 10000
N_EDGES = 320000
D_IN = 128
D_HID = 64
D1 = 80
D2 = 16

NC = 2
NS = 16
NW = NC * NS
E_PER_TILE = N_EDGES // NW
CHUNK = 80
STEPS = E_PER_TILE // CHUNK
ROWS_PER_TILE = N_NODES // NS

_TC_BLK = 1000
_TC_GRID = N_NODES // _TC_BLK


def _sc_segment_sum(table, src3d, dst3d, zrows, d):
    mesh = plsc.VectorSubcoreMesh(core_axis_name="c", subcore_axis_name="s")

    @functools.partial(
        pl.kernel,
        out_type=jax.ShapeDtypeStruct((NC, NS, ROWS_PER_TILE, d), jnp.float32),
        mesh=mesh,
        compiler_params=pltpu.CompilerParams(use_tc_tiling_on_sc=False),
        scratch_types=[
            pltpu.VMEM((STEPS, CHUNK), jnp.int32),
            pltpu.VMEM((STEPS, CHUNK), jnp.int32),
            pltpu.VMEM((CHUNK, d), jnp.float32),
            pltpu.VMEM_SHARED((N_NODES, d), jnp.float32),
            pltpu.SemaphoreType.DMA,
        ],
    )
    def k(table_hbm, src_hbm, dst_hbm, z_hbm, out_hbm, src_v, dst_v, rows_v, acc_sh, sem):
        c = lax.axis_index("c")
        s = lax.axis_index("s")
        wid = c * NS + s
        pltpu.sync_copy(z_hbm, acc_sh.at[pl.ds(s * ROWS_PER_TILE, ROWS_PER_TILE)])
        pltpu.sync_copy(src_hbm.at[wid], src_v)
        pltpu.sync_copy(dst_hbm.at[wid], dst_v)
        plsc.subcore_barrier()

        @pl.loop(0, STEPS)
        def _(i):
            pltpu.async_copy(table_hbm.at[src_v.at[i]], rows_v, sem).wait()
            pltpu.sync_copy(rows_v, acc_sh.at[dst_v.at[i]], add=True)

        plsc.subcore_barrier()
        pltpu.sync_copy(
            acc_sh.at[pl.ds(s * ROWS_PER_TILE, ROWS_PER_TILE)],
            out_hbm.at[c, s],
        )

    return k(table, src3d, dst3d, zrows).reshape(NC, N_NODES, d)


def _tc_project1(x, wl_pad, wr, crow):
    def body(x_ref, wl_ref, wr_ref, c_ref, p_ref, r_ref):
        xx = x_ref[...]
        p_ref[...] = jnp.dot(xx, wl_ref[...], preferred_element_type=jnp.float32) + c_ref[...]
        r_ref[...] = jnp.dot(xx, wr_ref[...], preferred_element_type=jnp.float32)

    return pl.pallas_call(
        body,
        grid=(_TC_GRID,),
        in_specs=[
            pl.BlockSpec((_TC_BLK, D_IN), lambda i: (i, 0)),
            pl.BlockSpec((D_IN, D1), lambda i: (0, 0)),
            pl.BlockSpec((D_IN, D_HID), lambda i: (0, 0)),
            pl.BlockSpec((1, D1), lambda i: (0, 0)),
        ],
        out_specs=[
            pl.BlockSpec((_TC_BLK, D1), lambda i: (i, 0)),
            pl.BlockSpec((_TC_BLK, D_HID), lambda i: (i, 0)),
        ],
        out_shape=[
            jax.ShapeDtypeStruct((N_NODES, D1), jnp.float32),
            jax.ShapeDtypeStruct((N_NODES, D_HID), jnp.float32),
        ],
    )(x, wl_pad, wr, crow)


def _tc_middle(agg, r1, b1row, w2l_pad, w2r):
    def body(a_ref, r1_ref, b1_ref, wl_ref, wr_ref, q_ref, r2_ref, deg_ref):
        a = a_ref[0] + a_ref[1]
        deg = a[:, D_HID:D_HID + 1]
        degc = jnp.maximum(deg, 1.0)
        h = jnp.maximum(a[:, :D_HID] / degc + b1_ref[...] + r1_ref[...], 0.0)
        q_ref[...] = jnp.dot(h, wl_ref[...], preferred_element_type=jnp.float32)
        r2_ref[...] = jnp.dot(h, wr_ref[...], preferred_element_type=jnp.float32)
        deg_ref[...] = deg

    return pl.pallas_call(
        body,
        grid=(_TC_GRID,),
        in_specs=[
            pl.BlockSpec((NC, _TC_BLK, D1), lambda i: (0, i, 0)),
            pl.BlockSpec((_TC_BLK, D_HID), lambda i: (i, 0)),
            pl.BlockSpec((1, D_HID), lambda i: (0, 0)),
            pl.BlockSpec((D_HID, D2), lambda i: (0, 0)),
            pl.BlockSpec((D_HID, 1), lambda i: (0, 0)),
        ],
        out_specs=[
            pl.BlockSpec((_TC_BLK, D2), lambda i: (i, 0)),
            pl.BlockSpec((_TC_BLK, 1), lambda i: (i, 0)),
            pl.BlockSpec((_TC_BLK, 1), lambda i: (i, 0)),
        ],
        out_shape=[
            jax.ShapeDtypeStruct((N_NODES, D2), jnp.float32),
            jax.ShapeDtypeStruct((N_NODES, 1), jnp.float32),
            jax.ShapeDtypeStruct((N_NODES, 1), jnp.float32),
        ],
    )(agg, r1, b1row, w2l_pad, w2r)


def _tc_final(agg2, deg, r2, b2row):
    def body(q_ref, deg_ref, r2_ref, b2_ref, o_ref):
        q = q_ref[0] + q_ref[1]
        s = q[:, 0:1]
        o_ref[...] = s / jnp.maximum(deg_ref[...], 1.0) + b2_ref[...] + r2_ref[...]

    return pl.pallas_call(
        body,
        grid=(_TC_GRID,),
        in_specs=[
            pl.BlockSpec((NC, _TC_BLK, D2), lambda i: (0, i, 0)),
            pl.BlockSpec((_TC_BLK, 1), lambda i: (i, 0)),
            pl.BlockSpec((_TC_BLK, 1), lambda i: (i, 0)),
            pl.BlockSpec((1, 1), lambda i: (0, 0)),
        ],
        out_specs=pl.BlockSpec((_TC_BLK, 1), lambda i: (i, 0)),
        out_shape=jax.ShapeDtypeStruct((N_NODES, 1), jnp.float32),
    )(agg2, deg, r2, b2row)


def kernel(x, edge_index, W1_l, b1, W1_r, W2_l, b2, W2_r):
    src3d = edge_index[0].astype(jnp.int32).reshape(NW, STEPS, CHUNK)
    dst3d = edge_index[1].astype(jnp.int32).reshape(NW, STEPS, CHUNK)

    wl_pad = jnp.concatenate([W1_l, jnp.zeros((D_IN, D1 - D_HID), jnp.float32)], axis=1)
    crow = jnp.zeros((1, D1), jnp.float32).at[0, D_HID].set(1.0)
    w2l_pad = jnp.concatenate([W2_l, jnp.zeros((D_HID, D2 - 1), jnp.float32)], axis=1)

    p_tab, r1 = _tc_project1(x, wl_pad, W1_r, crow)
    agg1 = _sc_segment_sum(p_tab, src3d, dst3d,
                           jnp.zeros((ROWS_PER_TILE, D1), jnp.float32), D1)
    q_tab, r2, deg = _tc_middle(agg1, r1, b1.reshape(1, D_HID), w2l_pad,
                                W2_r.reshape(D_HID, 1))
    agg2 = _sc_segment_sum(q_tab, src3d, dst3d,
                           jnp.zeros((ROWS_PER_TILE, D2), jnp.float32), D2)
    return _tc_final(agg2, deg, r2, b2.reshape(1, 1))
